# baseline (speedup 1.0000x reference)
import jax
import jax.numpy as jnp
from jax import lax
from jax.experimental import pallas as pl


def _body(x_ref, k_ref, o_ref, *, F, A, stride):
    s = jnp.sum(x_ref[...], axis=1)
    row = lax.broadcasted_iota(jnp.int32, (F, A), 0)
    col = lax.broadcasted_iota(jnp.int32, (F, A), 1)
    sel = jnp.where(row == col * stride, -k_ref[...], 0.0)
    o_ref[...] = jnp.dot(s, sel, preferred_element_type=jnp.float32)


def kernel(inputs, kernel):
    B, L, F, _ = inputs.shape
    A = kernel.shape[2]
    stride = F // A
    x = inputs.reshape(B, L, F)
    k = kernel.reshape(1, A)
    Bt = 32
    import functools
    body = functools.partial(_body, F=F, A=A, stride=stride)
    out = pl.pallas_call(
        body,
        grid=(B // Bt,),
        in_specs=[
            pl.BlockSpec((Bt, L, F), lambda i: (i, 0, 0)),
            pl.BlockSpec((1, A), lambda i: (0, 0)),
        ],
        out_specs=pl.BlockSpec((Bt, A), lambda i: (i, 0)),
        out_shape=jax.ShapeDtypeStruct((B, A), jnp.float32),
    )(x, k)
    return out.reshape(B, A, 1)
